# baseline (speedup 1.0000x reference)
import functools

import jax
import jax.numpy as jnp
from jax import lax
from jax.experimental import pallas as pl
from jax.experimental.pallas import tpu as pltpu
from jax.experimental.pallas import tpu_sc as plsc

_CHUNK = 128


def _lookup_t(idx2, table_t, *, d, batch):
    n_chunks = batch // _CHUNK
    mesh = plsc.VectorSubcoreMesh(core_axis_name="c", subcore_axis_name="s")
    nc = plsc.get_sparse_core_info().num_cores

    @functools.partial(
        pl.kernel,
        mesh=mesh,
        compiler_params=pltpu.CompilerParams(use_tc_tiling_on_sc=False),
        out_type=jax.ShapeDtypeStruct((d, batch), jnp.float32),
        scratch_types=[
            pltpu.VMEM((n_chunks, _CHUNK), jnp.int32),
            pltpu.VMEM((batch,), jnp.float32),
            pltpu.SemaphoreType.DMA,
        ],
    )
    def k(idx_hbm, tab_hbm, out_hbm, idx_v, row_v, sem):
        wid = lax.axis_index("s") * nc + lax.axis_index("c")
        pltpu.sync_copy(idx_hbm, idx_v)

        def fire(kc, carry):
            off = pl.multiple_of(kc * _CHUNK, _CHUNK)
            pltpu.async_copy(
                tab_hbm.at[wid].at[idx_v.at[kc]],
                row_v.at[pl.ds(off, _CHUNK)],
                sem,
            )
            return carry

        lax.fori_loop(0, n_chunks, fire, 0)
        pltpu.make_async_copy(out_hbm.at[wid], row_v, sem).wait()
        pltpu.sync_copy(row_v, out_hbm.at[wid])

    return k(idx2, table_t)


def kernel(x, emb_table):
    (batch,) = x.shape
    _, d = emb_table.shape
    table_t = emb_table.T
    idx2 = x.astype(jnp.int32).reshape(batch // _CHUNK, _CHUNK)
    out_t = _lookup_t(idx2, table_t, d=d, batch=batch)
    return out_t.T


# speedup vs baseline: 4.9536x; 4.9536x over previous
import functools

import jax
import jax.numpy as jnp
from jax import lax
from jax.experimental import pallas as pl
from jax.experimental.pallas import tpu as pltpu
from jax.experimental.pallas import tpu_sc as plsc

_CHUNK = 128


def _lookup(idx3, table, *, d, batch):
    mesh = plsc.VectorSubcoreMesh(core_axis_name="c", subcore_axis_name="s")
    info = plsc.get_sparse_core_info()
    nc, ns = info.num_cores, info.num_subcores
    nw = nc * ns
    b_per_w = batch // nw
    n_chunks = b_per_w // _CHUNK

    @functools.partial(
        pl.kernel,
        mesh=mesh,
        compiler_params=pltpu.CompilerParams(use_tc_tiling_on_sc=False),
        out_type=jax.ShapeDtypeStruct((batch, d), jnp.float32),
        scratch_types=[
            pltpu.VMEM((n_chunks, _CHUNK), jnp.int32),
            pltpu.VMEM((b_per_w, d), jnp.float32),
            pltpu.SemaphoreType.DMA,
        ],
    )
    def k(idx_hbm, tab_hbm, out_hbm, idx_v, rows_v, sem):
        wid = lax.axis_index("s") * nc + lax.axis_index("c")
        base = wid * b_per_w
        pltpu.sync_copy(idx_hbm.at[wid], idx_v)

        def fire(kc, carry):
            off = pl.multiple_of(kc * _CHUNK, _CHUNK)
            pltpu.async_copy(
                tab_hbm.at[idx_v.at[kc]],
                rows_v.at[pl.ds(off, _CHUNK)],
                sem,
            )
            return carry

        lax.fori_loop(0, n_chunks, fire, 0)
        pltpu.make_async_copy(out_hbm.at[pl.ds(base, b_per_w)], rows_v, sem).wait()
        pltpu.sync_copy(rows_v, out_hbm.at[pl.ds(base, b_per_w)])

    return k(idx3, table)


def kernel(x, emb_table):
    (batch,) = x.shape
    _, d = emb_table.shape
    info = plsc.get_sparse_core_info()
    nw = info.num_cores * info.num_subcores
    idx3 = x.astype(jnp.int32).reshape(nw, batch // (nw * _CHUNK), _CHUNK)
    return _lookup(idx3, emb_table, d=d, batch=batch)
